# baseline (speedup 1.0000x reference)
import jax
import jax.numpy as jnp
from jax.experimental import pallas as pl
from jax.experimental.pallas import tpu as pltpu

B_BLK = 16


def _body(x_ref, t_ref, o_ref):
    t = t_ref[...]
    norms = jnp.sqrt(jnp.sum(t * t, axis=-1, keepdims=True))
    scale = jnp.where(norms > 1.0, 1.0 / (norms + 1e-7), 1.0)
    o_ref[...] = x_ref[...] + t * scale


def kernel(x, table):
    B, S, one, D = x.shape
    x3 = x.reshape(B, S, D)
    grid = (B // B_BLK,)
    out = pl.pallas_call(
        _body,
        grid=grid,
        in_specs=[
            pl.BlockSpec((B_BLK, S, D), lambda i: (i, 0, 0)),
            pl.BlockSpec((S, D), lambda i: (0, 0)),
        ],
        out_specs=pl.BlockSpec((B_BLK, S, D), lambda i: (i, 0, 0)),
        out_shape=jax.ShapeDtypeStruct((B, S, D), x.dtype),
        compiler_params=pltpu.CompilerParams(
            dimension_semantics=("arbitrary",),
        ),
    )(x3, table)
    return out.reshape(B, S, one, D)


# speedup vs baseline: 1.2397x; 1.2397x over previous
import jax
import jax.numpy as jnp
from jax.experimental import pallas as pl
from jax.experimental.pallas import tpu as pltpu

B_BLK = 64


def _body(x_ref, t_ref, o_ref):
    t = t_ref[...]
    norms = jnp.sqrt(jnp.sum(t * t, axis=-1, keepdims=True))
    scale = jnp.where(norms > 1.0, 1.0 / (norms + 1e-7), 1.0)
    o_ref[...] = x_ref[...] + t * scale


def kernel(x, table):
    B, S, one, D = x.shape
    x3 = x.reshape(B, S, D)
    grid = (B // B_BLK,)
    out = pl.pallas_call(
        _body,
        grid=grid,
        in_specs=[
            pl.BlockSpec((B_BLK, S, D), lambda i: (i, 0, 0)),
            pl.BlockSpec((S, D), lambda i: (0, 0)),
        ],
        out_specs=pl.BlockSpec((B_BLK, S, D), lambda i: (i, 0, 0)),
        out_shape=jax.ShapeDtypeStruct((B, S, D), x.dtype),
        compiler_params=pltpu.CompilerParams(
            dimension_semantics=("arbitrary",),
        ),
    )(x3, table)
    return out.reshape(B, S, one, D)


# speedup vs baseline: 1.2574x; 1.0143x over previous
import jax
import jax.numpy as jnp
from jax.experimental import pallas as pl
from jax.experimental.pallas import tpu as pltpu

B_BLK = 128


def _body(x_ref, t_ref, o_ref):
    t = t_ref[...]
    norms = jnp.sqrt(jnp.sum(t * t, axis=-1, keepdims=True))
    scale = jnp.where(norms > 1.0, 1.0 / (norms + 1e-7), 1.0)
    o_ref[...] = x_ref[...] + t * scale


def kernel(x, table):
    B, S, one, D = x.shape
    x3 = x.reshape(B, S, D)
    grid = (B // B_BLK,)
    out = pl.pallas_call(
        _body,
        grid=grid,
        in_specs=[
            pl.BlockSpec((B_BLK, S, D), lambda i: (i, 0, 0)),
            pl.BlockSpec((S, D), lambda i: (0, 0)),
        ],
        out_specs=pl.BlockSpec((B_BLK, S, D), lambda i: (i, 0, 0)),
        out_shape=jax.ShapeDtypeStruct((B, S, D), x.dtype),
        compiler_params=pltpu.CompilerParams(
            dimension_semantics=("arbitrary",),
        ),
    )(x3, table)
    return out.reshape(B, S, one, D)
